# Initial kernel scaffold; baseline (speedup 1.0000x reference)
import jax
import jax.numpy as jnp
from jax.experimental import pallas as pl


def kernel(actions, table):
    raise NotImplementedError("write your pallas kernel here")


# speedup vs baseline: 6.1344x; 6.1344x over previous
import functools

import jax
import jax.numpy as jnp
from jax import lax
from jax.experimental import pallas as pl
from jax.experimental.pallas import tpu as pltpu
from jax.experimental.pallas import tpu_sc as plsc

_EMBED_DIM = 32
_TOTAL = 16384 * 200

_info = plsc.get_sparse_core_info()
_NC, _NS = _info.num_cores, _info.num_subcores
_NW = _NC * _NS
_PER_W = _TOTAL // _NW
_CHUNK = 1024
_NCHUNK = _PER_W // _CHUNK

_mesh = plsc.VectorSubcoreMesh(core_axis_name="c", subcore_axis_name="s")


@functools.partial(
    pl.kernel,
    mesh=_mesh,
    out_type=jax.ShapeDtypeStruct((_TOTAL, _EMBED_DIM), jnp.float32),
    scratch_types=[
        pltpu.VMEM((_CHUNK,), jnp.int32),
        pltpu.VMEM((_CHUNK, _EMBED_DIM), jnp.float32),
        pltpu.SemaphoreType.DMA,
    ],
    compiler_params=pltpu.CompilerParams(use_tc_tiling_on_sc=False),
)
def _gather_all(idx_hbm, table_hbm, out_hbm, idx_v, rows_v, sem):
    wid = lax.axis_index("s") * _NC + lax.axis_index("c")
    base = wid * _PER_W

    def body(c, carry):
        off = base + c * _CHUNK
        pltpu.sync_copy(idx_hbm.at[pl.ds(off, _CHUNK)], idx_v)
        pltpu.async_copy(table_hbm.at[idx_v], rows_v, sem).wait()
        pltpu.sync_copy(rows_v, out_hbm.at[pl.ds(off, _CHUNK)])
        return carry

    lax.fori_loop(0, _NCHUNK, body, 0)


def kernel(actions, table):
    flat = actions.reshape(_TOTAL).astype(jnp.int32)
    out = _gather_all(flat, table)
    return out.reshape(actions.shape + (table.shape[1],))


# speedup vs baseline: 6.4831x; 1.0569x over previous
import functools

import jax
import jax.numpy as jnp
from jax import lax
from jax.experimental import pallas as pl
from jax.experimental.pallas import tpu as pltpu
from jax.experimental.pallas import tpu_sc as plsc

_EMBED_DIM = 32
_TOTAL = 16384 * 200

_info = plsc.get_sparse_core_info()
_NC, _NS = _info.num_cores, _info.num_subcores
_NW = _NC * _NS
_PER_W = _TOTAL // _NW
_CHUNK = 800
_NBUF = 4
_NCHUNK = _PER_W // _CHUNK
_NGROUP = _NCHUNK // _NBUF

_mesh = plsc.VectorSubcoreMesh(core_axis_name="c", subcore_axis_name="s")


@functools.partial(
    pl.kernel,
    mesh=_mesh,
    out_type=jax.ShapeDtypeStruct((_TOTAL, _EMBED_DIM), jnp.float32),
    scratch_types=[
        pltpu.VMEM((_NBUF, _CHUNK), jnp.int32),
        pltpu.VMEM((_NBUF, _CHUNK, _EMBED_DIM), jnp.float32),
        pltpu.SemaphoreType.DMA,
        pltpu.SemaphoreType.DMA,
        pltpu.SemaphoreType.DMA,
    ],
    compiler_params=pltpu.CompilerParams(use_tc_tiling_on_sc=False),
)
def _gather_all(idx_hbm, table_hbm, out_hbm, idx_v, rows_v, isem, gsem, osem):
    wid = lax.axis_index("s") * _NC + lax.axis_index("c")
    base = wid * _PER_W

    def idx_cp(c, b):
        return pltpu.make_async_copy(
            idx_hbm.at[pl.ds(base + c * _CHUNK, _CHUNK)], idx_v.at[b], isem)

    def gat_cp(b):
        return pltpu.make_async_copy(table_hbm.at[idx_v.at[b]], rows_v.at[b], gsem)

    def out_cp(c, b):
        return pltpu.make_async_copy(
            rows_v.at[b], out_hbm.at[pl.ds(base + c * _CHUNK, _CHUNK)], osem)

    for b in range(_NBUF):
        idx_cp(b, b).start()
    idx_cp(0, 0).wait()
    gat_cp(0).start()

    def body(g, carry):
        for j in range(_NBUF):
            c = g * _NBUF + j
            b = j
            bp = (j - 1) % _NBUF

            @pl.when(c > 0)
            def _():
                idx_cp(c, b).wait()

                @pl.when(c >= _NBUF)
                def _():
                    out_cp(c - _NBUF, b).wait()

                gat_cp(b).start()

            @pl.when(c > 0)
            def _():
                gat_cp(bp).wait()
                out_cp(c - 1, bp).start()

                @pl.when(c - 1 + _NBUF < _NCHUNK)
                def _():
                    idx_cp(c - 1 + _NBUF, bp).start()

        return carry

    lax.fori_loop(0, _NGROUP, body, 0)

    last_b = (_NCHUNK - 1) % _NBUF
    gat_cp(last_b).wait()
    out_cp(_NCHUNK - 1, last_b).start()
    for k in range(_NBUF):
        c = _NCHUNK - _NBUF + k
        out_cp(c, c % _NBUF).wait()


def kernel(actions, table):
    flat = actions.reshape(_TOTAL).astype(jnp.int32)
    out = _gather_all(flat, table)
    return out.reshape(actions.shape + (table.shape[1],))
